# baseline (speedup 1.0000x reference)
import functools

import jax
import jax.numpy as jnp
from jax import lax
from jax.experimental import pallas as pl
from jax.experimental.pallas import tpu as pltpu
from jax.experimental.pallas import tpu_sc as plsc

_CHH = 5
_NB = 2


def _build_gather(B, H, V, D, NC, NS):
    NW = NC * NS
    JB = B // 128
    assert JB == NW and D == 32 and H % _CHH == 0
    NCH = H // _CHH
    CH = _CHH * 128
    assert NCH % 2 == 0 and NCH >= 4
    mesh = plsc.VectorSubcoreMesh(core_axis_name="c", subcore_axis_name="s")

    @functools.partial(
        pl.kernel,
        mesh=mesh,
        out_type=jax.ShapeDtypeStruct((H, 4, JB, 8, 128), jnp.float32),
        scratch_types=[
            pltpu.VMEM((128, H), jnp.int32),
            pltpu.VMEM((_NB, CH), jnp.int32),
            pltpu.VMEM((_NB, CH, D), jnp.float32),
            pltpu.VMEM((_NB, _CHH, 4, 8, 128), jnp.float32),
            pltpu.SemaphoreType.DMA,
        ] + [pltpu.SemaphoreType.DMA] * (2 * _NB),
        compiler_params=pltpu.CompilerParams(
            use_tc_tiling_on_sc=False, needs_layout_passes=False),
    )
    def gather(idx_hbm, tbl_hbm, out_hbm, idx_raw, glist, gath, tbuf,
               sem_idx, *sems):
        sem_g = sems[0:_NB]
        sem_w = sems[_NB:2 * _NB]
        wid = lax.axis_index("s") * NC + lax.axis_index("c")
        iota = lax.iota(jnp.int32, 16)

        def build_glist(i, b):
            h0 = i * _CHH

            def g2body(g2, carry):
                jv = iota + g2 * 16
                for hl in range(_CHH):
                    hv = jnp.zeros((16,), jnp.int32) + (h0 + hl)
                    v = plsc.load_gather(idx_raw, [jv, hv])
                    glist[b, pl.ds(hl * 128 + g2 * 16, 16)] = v
                return carry

            lax.fori_loop(0, 8, g2body, 0)

        def issue_gather(b):
            pltpu.async_copy(tbl_hbm.at[glist.at[b]], gath.at[b], sem_g[b])

        def wait_gather(b):
            pltpu.make_async_copy(
                tbl_hbm.at[glist.at[b]], gath.at[b], sem_g[b]).wait()

        def transpose(b):
            def tbody(g2, carry):
                hl = lax.shift_right_logical(g2, 3)
                j16 = (g2 & 7) * 16
                rows = iota + (hl * 128 + j16)
                for c in range(D):
                    cols = jnp.full((16,), c, jnp.int32)
                    v = plsc.load_gather(gath.at[b], [rows, cols])
                    tbuf[b, hl, c // 8, c % 8, pl.ds(j16, 16)] = v
                return carry

            lax.fori_loop(0, _CHH * 8, tbody, 0)

        def issue_write(i, b):
            pltpu.async_copy(
                tbuf.at[b], out_hbm.at[pl.ds(i * _CHH, _CHH), :, wid],
                sem_w[b])

        def wait_write(b):
            pltpu.make_async_copy(
                tbuf.at[b], out_hbm.at[pl.ds(0, _CHH), :, wid],
                sem_w[b]).wait()

        pltpu.async_copy(
            idx_hbm.at[pl.ds(128 * wid, 128)], idx_raw, sem_idx).wait()

        for b in range(_NB):
            build_glist(b, b)
            issue_gather(b)

        for b in range(_NB):
            wait_gather(b)
            transpose(b)
            issue_write(b, b)
            build_glist(b + 2, b)
            issue_gather(b)

        def body(k, carry):
            for b in range(_NB):
                i = 2 * k + b
                wait_gather(b)
                wait_write(b)
                transpose(b)
                issue_write(i, b)
                build_glist(i + 2, b)
                issue_gather(b)
            return carry

        lax.fori_loop(1, NCH // 2 - 1, body, 0)

        for b in range(_NB):
            i = NCH - 2 + b
            wait_gather(b)
            wait_write(b)
            transpose(b)
            issue_write(i, b)
        for b in range(_NB):
            wait_write(b)

    return gather


def kernel(src, weight):
    B, H = src.shape
    V, D = weight.shape
    info = plsc.get_sparse_core_info()
    gather = _build_gather(B, H, V, D, info.num_cores, info.num_subcores)
    z = gather(src, weight)
    return z.transpose(2, 4, 0, 1, 3).reshape(B, H, D)


# speedup vs baseline: 1.5721x; 1.5721x over previous
import functools

import jax
import jax.numpy as jnp
from jax import lax
from jax.experimental import pallas as pl
from jax.experimental.pallas import tpu as pltpu
from jax.experimental.pallas import tpu_sc as plsc

_CHH = 5
_NB = 2


def _build_gather(B, H, V, D, NC, NS):
    NW = NC * NS
    JB = B // 128
    assert JB == NW and D == 32 and H % _CHH == 0
    NCH = H // _CHH
    CH = _CHH * 128
    assert NCH % 2 == 0 and NCH >= 4
    mesh = plsc.VectorSubcoreMesh(core_axis_name="c", subcore_axis_name="s")

    @functools.partial(
        pl.kernel,
        mesh=mesh,
        out_type=jax.ShapeDtypeStruct((H, 4, JB, 8, 128), jnp.float32),
        scratch_types=[
            pltpu.VMEM((128, H), jnp.int32),
            pltpu.VMEM((_NB, CH), jnp.int32),
            pltpu.VMEM((_NB, CH, D), jnp.float32),
            pltpu.VMEM((_NB, _CHH, 4, 8, 129), jnp.float32),
            pltpu.SemaphoreType.DMA,
        ] + [pltpu.SemaphoreType.DMA] * (2 * _NB),
        compiler_params=pltpu.CompilerParams(
            use_tc_tiling_on_sc=False, needs_layout_passes=False),
    )
    def gather(idx_hbm, tbl_hbm, out_hbm, idx_raw, glist, gath, tbuf,
               sem_idx, *sems):
        sem_g = sems[0:_NB]
        sem_w = sems[_NB:2 * _NB]
        wid = lax.axis_index("s") * NC + lax.axis_index("c")
        iota = lax.iota(jnp.int32, 16)

        def build_glist(i, b):
            h0 = i * _CHH

            def g2body(g2, carry):
                jv = iota + g2 * 16
                for hl in range(_CHH):
                    hv = jnp.zeros((16,), jnp.int32) + (h0 + hl)
                    v = plsc.load_gather(idx_raw, [jv, hv])
                    glist[b, pl.ds(hl * 128 + g2 * 16, 16)] = v
                return carry

            lax.fori_loop(0, 8, g2body, 0)

        def issue_gather(b):
            pltpu.async_copy(tbl_hbm.at[glist.at[b]], gath.at[b], sem_g[b])

        def wait_gather(b):
            pltpu.make_async_copy(
                tbl_hbm.at[glist.at[b]], gath.at[b], sem_g[b]).wait()

        def transpose(b):
            def tbody(j, carry):
                jv = jnp.zeros((16,), jnp.int32) + j
                for hl in range(_CHH):
                    for o in range(2):
                        v = gath[b, hl * 128 + j, pl.ds(o * 16, 16)]
                        cvec = iota + (o * 16)
                        plsc.store_scatter(
                            tbuf.at[b],
                            [jnp.full((16,), hl, jnp.int32),
                             lax.shift_right_logical(cvec, 3),
                             cvec & 7,
                             jv],
                            v)
                return carry

            lax.fori_loop(0, 128, tbody, 0)

        def issue_write(i, b):
            pltpu.async_copy(
                tbuf.at[b, :, :, :, pl.ds(0, 128)],
                out_hbm.at[pl.ds(i * _CHH, _CHH), :, wid], sem_w[b])

        def wait_write(b):
            pltpu.make_async_copy(
                tbuf.at[b, :, :, :, pl.ds(0, 128)],
                out_hbm.at[pl.ds(0, _CHH), :, wid], sem_w[b]).wait()

        pltpu.async_copy(
            idx_hbm.at[pl.ds(128 * wid, 128)], idx_raw, sem_idx).wait()

        for b in range(_NB):
            build_glist(b, b)
            issue_gather(b)

        for b in range(_NB):
            wait_gather(b)
            transpose(b)
            issue_write(b, b)
            build_glist(b + 2, b)
            issue_gather(b)

        def body(k, carry):
            for b in range(_NB):
                i = 2 * k + b
                wait_gather(b)
                wait_write(b)
                transpose(b)
                issue_write(i, b)
                build_glist(i + 2, b)
                issue_gather(b)
            return carry

        lax.fori_loop(1, NCH // 2 - 1, body, 0)

        for b in range(_NB):
            i = NCH - 2 + b
            wait_gather(b)
            wait_write(b)
            transpose(b)
            issue_write(i, b)
        for b in range(_NB):
            wait_write(b)

    return gather


def kernel(src, weight):
    B, H = src.shape
    V, D = weight.shape
    info = plsc.get_sparse_core_info()
    gather = _build_gather(B, H, V, D, info.num_cores, info.num_subcores)
    z = gather(src, weight)
    return z.transpose(2, 4, 0, 1, 3).reshape(B, H, D)
